# baseline (speedup 1.0000x reference)
import jax
import jax.numpy as jnp
from jax import lax
from jax.experimental import pallas as pl
from jax.experimental.pallas import tpu as pltpu
from jax.experimental.pallas import tpu_sc as plsc

D_MODEL = 2048
N_EXPERTS = 16
TOP_K = 2

CHUNK = 512
NBUF = 8

SC_CORES = 2
SC_SUBCORES = 16
SC_TILES = SC_CORES * SC_SUBCORES
SC_LANES = 16


def _router_tc_kernel(x_hbm, w_ref, probs_ref, buf, sem):
    i = pl.program_id(0)
    nchunk = pl.num_programs(0)

    def issue(c):
        slot = jax.lax.rem(c, NBUF)
        pltpu.make_async_copy(
            x_hbm.at[pl.ds(c * CHUNK, CHUNK), :], buf.at[slot], sem.at[slot]
        ).start()

    @pl.when(i == 0)
    def _prologue():
        for c in range(NBUF):
            issue(jnp.int32(c))

    slot = jax.lax.rem(i, NBUF)
    pltpu.make_async_copy(
        x_hbm.at[pl.ds(i * CHUNK, CHUNK), :], buf.at[slot], sem.at[slot]
    ).wait()

    x = buf[slot]
    w = w_ref[...]
    logits = jax.lax.dot_general(
        w, x, (((1,), (1,)), ((), ())), preferred_element_type=jnp.float32
    )

    m = jnp.max(logits, axis=0, keepdims=True)
    e = jnp.exp(logits - m)
    z = jnp.sum(e, axis=0, keepdims=True)
    probs_ref[...] = e / z

    @pl.when(i + NBUF < nchunk)
    def _lookahead():
        issue(i + NBUF)


def _sc_route_kernel(probs_hbm, idx_hbm, wgt_hbm, pbuf, ibuf, wbuf):
    n = probs_hbm.shape[1]
    tok_per_tile = n // SC_TILES
    wid = lax.axis_index("s") * SC_CORES + lax.axis_index("c")
    base = wid * tok_per_tile
    pltpu.sync_copy(probs_hbm.at[:, pl.ds(base, tok_per_tile)], pbuf)

    def body(g, carry):
        col = g * SC_LANES
        vals = [pbuf[e, pl.ds(col, SC_LANES)] for e in range(N_EXPERTS)]

        best = vals[0]
        bidx = jnp.zeros((SC_LANES,), jnp.int32)
        for e in range(1, N_EXPERTS):
            gt = vals[e] > best
            best = jnp.where(gt, vals[e], best)
            bidx = jnp.where(gt, jnp.full((SC_LANES,), e, jnp.int32), bidx)

        second = jnp.full((SC_LANES,), -jnp.inf, jnp.float32)
        sidx = jnp.zeros((SC_LANES,), jnp.int32)
        for e in range(N_EXPERTS):
            cand = jnp.where(bidx == e, -jnp.inf, vals[e])
            gt = cand > second
            second = jnp.where(gt, cand, second)
            sidx = jnp.where(gt, jnp.full((SC_LANES,), e, jnp.int32), sidx)

        denom = best + second + 1e-09
        ibuf[0, pl.ds(col, SC_LANES)] = bidx
        ibuf[1, pl.ds(col, SC_LANES)] = sidx
        wbuf[0, pl.ds(col, SC_LANES)] = best / denom
        wbuf[1, pl.ds(col, SC_LANES)] = second / denom
        return carry

    lax.fori_loop(0, tok_per_tile // SC_LANES, body, 0)

    pltpu.sync_copy(ibuf, idx_hbm.at[:, pl.ds(base, tok_per_tile)])
    pltpu.sync_copy(wbuf, wgt_hbm.at[:, pl.ds(base, tok_per_tile)])


def kernel(x, W):
    n = x.shape[0]
    grid = (n // CHUNK,)
    probs_t = pl.pallas_call(
        _router_tc_kernel,
        grid=grid,
        in_specs=[
            pl.BlockSpec(memory_space=pltpu.HBM),
            pl.BlockSpec((N_EXPERTS, D_MODEL), lambda i: (0, 0)),
        ],
        out_specs=pl.BlockSpec((N_EXPERTS, CHUNK), lambda i: (0, i)),
        out_shape=jax.ShapeDtypeStruct((N_EXPERTS, n), jnp.float32),
        scratch_shapes=[
            pltpu.VMEM((NBUF, CHUNK, D_MODEL), jnp.float32),
            pltpu.SemaphoreType.DMA((NBUF,)),
        ],
        compiler_params=pltpu.CompilerParams(
            dimension_semantics=("arbitrary",),
        ),
    )(x, W)

    tok_per_tile = n // SC_TILES
    mesh = plsc.VectorSubcoreMesh(core_axis_name="c", subcore_axis_name="s")
    idx_t, wgt_t = pl.kernel(
        _sc_route_kernel,
        out_type=[
            jax.ShapeDtypeStruct((8, n), jnp.int32),
            jax.ShapeDtypeStruct((8, n), jnp.float32),
        ],
        mesh=mesh,
        scratch_types=[
            pltpu.VMEM((N_EXPERTS, tok_per_tile), jnp.float32),
            pltpu.VMEM((8, tok_per_tile), jnp.int32),
            pltpu.VMEM((8, tok_per_tile), jnp.float32),
        ],
        compiler_params=pltpu.CompilerParams(use_tc_tiling_on_sc=True),
    )(probs_t)

    top_idx = idx_t[:TOP_K].T
    weights = wgt_t[:TOP_K].T
    probs = probs_t.T
    return (top_idx, weights, probs)


# speedup vs baseline: 1.7489x; 1.7489x over previous
import jax
import jax.numpy as jnp
from jax.experimental import pallas as pl
from jax.experimental.pallas import tpu as pltpu

D_MODEL = 2048
N_EXPERTS = 16
TOP_K = 2

CHUNK = 256
NBUF = 16


def _router_kernel(x_hbm, w_ref, gate_ref, probs_ref, buf, sem):
    i = pl.program_id(0)
    nchunk = pl.num_programs(0)

    def issue(c):
        slot = jax.lax.rem(c, NBUF)
        pltpu.make_async_copy(
            x_hbm.at[pl.ds(c * CHUNK, CHUNK), :], buf.at[slot], sem.at[slot]
        ).start()

    @pl.when(i == 0)
    def _prologue():
        for c in range(NBUF):
            issue(jnp.int32(c))

    slot = jax.lax.rem(i, NBUF)
    pltpu.make_async_copy(
        x_hbm.at[pl.ds(i * CHUNK, CHUNK), :], buf.at[slot], sem.at[slot]
    ).wait()

    x = buf[slot]
    w = w_ref[...]
    logits = jax.lax.dot_general(
        w, x, (((1,), (1,)), ((), ())), preferred_element_type=jnp.float32
    )

    m = jnp.max(logits, axis=0, keepdims=True)
    e = jnp.exp(logits - m)
    z = jnp.sum(e, axis=0, keepdims=True)
    probs = e / z
    probs_ref[...] = probs

    rows = jax.lax.broadcasted_iota(jnp.int32, probs.shape, 0)
    big = jnp.int32(N_EXPERTS)

    p1 = jnp.max(probs, axis=0, keepdims=True)
    i1 = jnp.min(jnp.where(probs >= p1, rows, big), axis=0, keepdims=True)
    masked = jnp.where(rows == i1, -jnp.inf, probs)
    p2 = jnp.max(masked, axis=0, keepdims=True)
    i2 = jnp.min(jnp.where(masked >= p2, rows, big), axis=0, keepdims=True)

    denom = p1 + p2 + 1e-09
    zero = jnp.zeros((4, CHUNK), jnp.float32)
    gate_ref[...] = jnp.concatenate(
        [i1.astype(jnp.float32), i2.astype(jnp.float32), p1 / denom, p2 / denom, zero],
        axis=0,
    )

    @pl.when(i + NBUF < nchunk)
    def _lookahead():
        issue(i + NBUF)


def kernel(x, W):
    n = x.shape[0]
    grid = (n // CHUNK,)
    out_shapes = (
        jax.ShapeDtypeStruct((8, n), jnp.float32),
        jax.ShapeDtypeStruct((N_EXPERTS, n), jnp.float32),
    )
    gate_t, probs_t = pl.pallas_call(
        _router_kernel,
        grid=grid,
        in_specs=[
            pl.BlockSpec(memory_space=pltpu.HBM),
            pl.BlockSpec((N_EXPERTS, D_MODEL), lambda i: (0, 0)),
        ],
        out_specs=(
            pl.BlockSpec((8, CHUNK), lambda i: (0, i)),
            pl.BlockSpec((N_EXPERTS, CHUNK), lambda i: (0, i)),
        ),
        out_shape=out_shapes,
        scratch_shapes=[
            pltpu.VMEM((NBUF, CHUNK, D_MODEL), jnp.float32),
            pltpu.SemaphoreType.DMA((NBUF,)),
        ],
        compiler_params=pltpu.CompilerParams(
            dimension_semantics=("arbitrary",),
        ),
    )(x, W)
    top_idx = gate_t[:TOP_K].T.astype(jnp.int32)
    weights = gate_t[TOP_K:2 * TOP_K].T
    probs = probs_t.T
    return (top_idx, weights, probs)
